# baseline (speedup 1.0000x reference)
import functools

import jax
import jax.numpy as jnp
from jax import lax
from jax.experimental import pallas as pl
from jax.experimental.pallas import tpu as pltpu
from jax.experimental.pallas import tpu_sc as plsc

N = 10000
E = 320000
F_IN = 128
HID = 64

NC = 2
NS = 16
NW = NC * NS
EPW = E // NW
CH = 80
NCHUNK = EPW // CH

_MESH = dict(core_axis_name="c", subcore_axis_name="s", num_cores=NC,
             num_subcores=NS)
_SC_PARAMS = pltpu.CompilerParams(needs_layout_passes=False,
                                  use_tc_tiling_on_sc=False)


def _worker_ids():
    c = lax.axis_index("c")
    s = lax.axis_index("s")
    return c, s, c * NS + s


def _writeback(c, s, acc, out0_hbm, out1_hbm):
    @pl.when(jnp.logical_and(s == 0, c == 0))
    def _():
        pltpu.sync_copy(acc, out0_hbm)

    @pl.when(jnp.logical_and(s == 0, c == 1))
    def _():
        pltpu.sync_copy(acc, out1_hbm)


DEG_Q = 8


def _deg_body(ew_hbm, row2d_hbm, zn_hbm, out0_hbm, out1_hbm, wall, rall, acc,
              dsem):
    c, s, w = _worker_ids()

    @pl.when(s == 0)
    def _():
        pltpu.sync_copy(zn_hbm, acc)

    pltpu.sync_copy(ew_hbm.at[pl.ds(w * EPW, EPW)], wall)
    pltpu.sync_copy(row2d_hbm.at[w], rall)
    plsc.subcore_barrier()

    def fire(k):
        pltpu.make_async_copy(
            wall.at[pl.ds(k * CH, CH)], acc.at[rall.at[k]], dsem
        ).start(add=True)

    def drain(k):
        pltpu.make_async_copy(
            wall.at[pl.ds(k * CH, CH)], acc.at[rall.at[k]], dsem
        ).wait()

    for k in range(DEG_Q):
        fire(k)

    def step(k, carry):
        drain(k - DEG_Q)
        fire(k)
        return carry

    lax.fori_loop(DEG_Q, NCHUNK, step, 0)
    for k in range(DEG_Q):
        drain(k)
    plsc.subcore_barrier()
    _writeback(c, s, acc, out0_hbm, out1_hbm)


def _deg_call(ew, row2d, zn):
    return pl.kernel(
        _deg_body,
        out_type=(jax.ShapeDtypeStruct((N,), jnp.float32),
                  jax.ShapeDtypeStruct((N,), jnp.float32)),
        mesh=plsc.VectorSubcoreMesh(**_MESH),
        compiler_params=_SC_PARAMS,
        scratch_types=[
            pltpu.VMEM((EPW,), jnp.float32),
            pltpu.VMEM((NCHUNK, CH), jnp.int32),
            pltpu.MemorySpace.VMEM_SHARED((N,), jnp.float32),
            pltpu.SemaphoreType.DMA,
        ],
    )(ew, row2d, zn)


NCH = 2000
NNCH = EPW // NCH


def _norm_body(row_hbm, col_hbm, ew_hbm, dinv_hbm, out_hbm,
               dbuf, rbuf, cbuf, wbuf, nbuf, sem):
    _, _, w = _worker_ids()
    pltpu.sync_copy(dinv_hbm, dbuf)

    def step(k, carry):
        off = w * EPW + k * NCH
        pltpu.sync_copy(row_hbm.at[pl.ds(off, NCH)], rbuf)
        pltpu.sync_copy(col_hbm.at[pl.ds(off, NCH)], cbuf)
        pltpu.sync_copy(ew_hbm.at[pl.ds(off, NCH)], wbuf)

        def grp(g, carry2):
            sl = pl.ds(g * 16, 16)
            dr = plsc.load_gather(dbuf, [rbuf[sl]])
            dc = plsc.load_gather(dbuf, [cbuf[sl]])
            nbuf[sl] = -(dr * wbuf[sl] * dc)
            return carry2

        lax.fori_loop(0, NCH // 16, grp, 0)
        pltpu.sync_copy(nbuf, out_hbm.at[pl.ds(off, NCH)])
        return carry

    lax.fori_loop(0, NNCH, step, 0)


def _norm_call(row, col, ew, dinv):
    return pl.kernel(
        _norm_body,
        out_type=jax.ShapeDtypeStruct((E,), jnp.float32),
        mesh=plsc.VectorSubcoreMesh(**_MESH),
        compiler_params=_SC_PARAMS,
        scratch_types=[
            pltpu.VMEM((N,), jnp.float32),
            pltpu.VMEM((NCH,), jnp.int32),
            pltpu.VMEM((NCH,), jnp.int32),
            pltpu.VMEM((NCH,), jnp.float32),
            pltpu.VMEM((NCH,), jnp.float32),
            pltpu.SemaphoreType.DMA,
        ],
    )(row, col, ew, dinv)


def _seg_body(F, xin_hbm, row2d_hbm, col2d_hbm, norm_hbm, z_hbm,
              out0_hbm, out1_hbm, rall, call, nall, dbuf0, dbuf1, acc,
              gsem0, gsem1, ssem0, ssem1):
    c, s, w = _worker_ids()

    @pl.when(s == 0)
    def _():
        pltpu.sync_copy(z_hbm, acc)

    pltpu.sync_copy(row2d_hbm.at[w], rall)
    pltpu.sync_copy(col2d_hbm.at[w], call)
    pltpu.sync_copy(norm_hbm.at[pl.ds(w * EPW, EPW)], nall)
    plsc.subcore_barrier()

    def g_start(k, dbuf, gsem):
        pltpu.make_async_copy(xin_hbm.at[rall.at[k]], dbuf, gsem).start()

    def g_wait(k, dbuf, gsem):
        pltpu.make_async_copy(xin_hbm.at[rall.at[k]], dbuf, gsem).wait()

    def s_start(k, dbuf, ssem):
        pltpu.make_async_copy(dbuf, acc.at[call.at[k]], ssem).start(add=True)

    def s_wait(k, dbuf, ssem):
        pltpu.make_async_copy(dbuf, acc.at[call.at[k]], ssem).wait()

    def scale(k, dbuf):
        @plsc.parallel_loop(0, CH // 16, 1, unroll=5)
        def _(g):
            n16 = nall[pl.ds(k * CH + g * 16, 16)]
            for j in range(16):
                spl = jnp.take_along_axis(
                    n16, jnp.full((16,), j, jnp.int32), axis=0)
                e = g * 16 + j
                for v in range(F // 16):
                    fs = pl.ds(v * 16, 16)
                    dbuf[e, fs] = dbuf[e, fs] * spl

    g_start(0, dbuf0, gsem0)

    def pair(p, carry):
        k0 = 2 * p
        @pl.when(p > 0)
        def _():
            s_wait(k0 - 1, dbuf1, ssem1)
        g_start(k0 + 1, dbuf1, gsem1)
        g_wait(k0, dbuf0, gsem0)
        scale(k0, dbuf0)
        s_start(k0, dbuf0, ssem0)
        s_wait(k0, dbuf0, ssem0)
        g_start(k0 + 2, dbuf0, gsem0)
        g_wait(k0 + 1, dbuf1, gsem1)
        scale(k0 + 1, dbuf1)
        s_start(k0 + 1, dbuf1, ssem1)
        return carry

    lax.fori_loop(0, (NCHUNK - 1) // 2, pair, 0)
    kl = NCHUNK - 1
    s_wait(kl - 1, dbuf1, ssem1)
    g_wait(kl, dbuf0, gsem0)
    scale(kl, dbuf0)
    s_start(kl, dbuf0, ssem0)
    s_wait(kl, dbuf0, ssem0)
    plsc.subcore_barrier()
    _writeback(c, s, acc, out0_hbm, out1_hbm)


def _seg_call(F, xin, row2d, col2d, norm, zeros2d):
    return pl.kernel(
        functools.partial(_seg_body, F),
        out_type=(jax.ShapeDtypeStruct((N, F), jnp.float32),
                  jax.ShapeDtypeStruct((N, F), jnp.float32)),
        mesh=plsc.VectorSubcoreMesh(**_MESH),
        compiler_params=_SC_PARAMS,
        scratch_types=[
            pltpu.VMEM((NCHUNK, CH), jnp.int32),
            pltpu.VMEM((NCHUNK, CH), jnp.int32),
            pltpu.VMEM((EPW,), jnp.float32),
            pltpu.VMEM((CH, F), jnp.float32),
            pltpu.VMEM((CH, F), jnp.float32),
            pltpu.MemorySpace.VMEM_SHARED((N, F), jnp.float32),
            pltpu.SemaphoreType.DMA,
            pltpu.SemaphoreType.DMA,
            pltpu.SemaphoreType.DMA,
            pltpu.SemaphoreType.DMA,
        ],
    )(xin, row2d, col2d, norm, zeros2d)


def _dinv_body(p0_ref, p1_ref, out_ref):
    deg = p0_ref[...] + p1_ref[...]
    out_ref[...] = jnp.where(
        deg > 0.0, lax.rsqrt(jnp.maximum(deg, 1e-12)), 0.0)


def _dinv_call(p0, p1):
    return pl.pallas_call(
        _dinv_body,
        out_shape=jax.ShapeDtypeStruct((1, N), jnp.float32),
    )(p0.reshape(1, N), p1.reshape(1, N))


MBLK = 2000


def _dense_body(head, x_ref, p0_ref, p1_ref, wz0_ref, wz1_ref, wh0_ref,
                wh1_ref, bz_ref, bh_ref, lw_ref, lb_ref, out_ref):
    t = p0_ref[...] + p1_ref[...]
    x = x_ref[...]
    a = (jnp.dot(x, wz0_ref[...], preferred_element_type=jnp.float32)
         + jnp.dot(t, wz1_ref[...], preferred_element_type=jnp.float32)
         + bz_ref[...])
    b = (jnp.dot(x, wh0_ref[...], preferred_element_type=jnp.float32)
         + jnp.dot(t, wh1_ref[...], preferred_element_type=jnp.float32)
         + bh_ref[...])
    h = (1.0 - jax.nn.sigmoid(a)) * jnp.tanh(b)
    if head:
        out_ref[...] = (jnp.dot(jax.nn.relu(h), lw_ref[...],
                                preferred_element_type=jnp.float32)
                        + lb_ref[...])
    else:
        out_ref[...] = h


def _dense_call(F, head, x, p0, p1, wz0, wz1, wh0, wh1, bz, bh, lw, lb):
    fout = 1 if head else HID
    return pl.pallas_call(
        functools.partial(_dense_body, head),
        grid=(N // MBLK,),
        in_specs=[
            pl.BlockSpec((MBLK, F), lambda i: (i, 0)),
            pl.BlockSpec((MBLK, F), lambda i: (i, 0)),
            pl.BlockSpec((MBLK, F), lambda i: (i, 0)),
            pl.BlockSpec((F, HID), lambda i: (0, 0)),
            pl.BlockSpec((F, HID), lambda i: (0, 0)),
            pl.BlockSpec((F, HID), lambda i: (0, 0)),
            pl.BlockSpec((F, HID), lambda i: (0, 0)),
            pl.BlockSpec((1, HID), lambda i: (0, 0)),
            pl.BlockSpec((1, HID), lambda i: (0, 0)),
            pl.BlockSpec((HID, 1), lambda i: (0, 0)),
            pl.BlockSpec((1, 1), lambda i: (0, 0)),
        ],
        out_specs=pl.BlockSpec((MBLK, fout), lambda i: (i, 0)),
        out_shape=jax.ShapeDtypeStruct((N, fout), jnp.float32),
    )(x, p0, p1, wz0, wz1, wh0, wh1, bz, bh, lw, lb)


def kernel(x, edge_index, edge_weight, W0x0, W0x1, b0x, W0h0, W0h1, b0h,
           W1x0, W1x1, b1x, W1h0, W1h1, b1h, lin_w, lin_b):
    row = edge_index[0].astype(jnp.int32)
    col = edge_index[1].astype(jnp.int32)
    ew = edge_weight.astype(jnp.float32)

    zn = jnp.zeros((N,), jnp.float32)
    z128 = jnp.zeros((N, F_IN), jnp.float32)
    z64 = jnp.zeros((N, HID), jnp.float32)

    row2d = row.reshape(NW, NCHUNK, CH)
    col2d = col.reshape(NW, NCHUNK, CH)

    d0, d1 = _deg_call(ew, row2d, zn)
    dinv = _dinv_call(d0, d1).reshape(N)
    norm = _norm_call(row, col, ew, dinv)

    bz0 = (b0x[0] + b0h[0]).reshape(1, HID)
    bh0 = (b0x[2] + b0h[2]).reshape(1, HID)
    bz1 = (b1x[0] + b1h[0]).reshape(1, HID)
    bh1 = (b1x[2] + b1h[2]).reshape(1, HID)
    lb = lin_b.reshape(1, 1)

    t0a, t0b = _seg_call(F_IN, x, row2d, col2d, norm, z128)
    h0 = _dense_call(F_IN, False, x, t0a, t0b, W0x0[0], W0x1[0], W0x0[2],
                     W0x1[2], bz0, bh0, lin_w, lb)
    t1a, t1b = _seg_call(HID, h0, row2d, col2d, norm, z64)
    out = _dense_call(HID, True, h0, t1a, t1b, W1x0[0], W1x1[0], W1x0[2],
                      W1x1[2], bz1, bh1, lin_w, lb)
    return out


# speedup vs baseline: 1.0062x; 1.0062x over previous
import functools

import jax
import jax.numpy as jnp
from jax import lax
from jax.experimental import pallas as pl
from jax.experimental.pallas import tpu as pltpu
from jax.experimental.pallas import tpu_sc as plsc

N = 10000
E = 320000
F_IN = 128
HID = 64

NC = 2
NS = 16
NW = NC * NS
EPW = E // NW
CH = 80
NCHUNK = EPW // CH

_MESH = dict(core_axis_name="c", subcore_axis_name="s", num_cores=NC,
             num_subcores=NS)
_SC_PARAMS = pltpu.CompilerParams(needs_layout_passes=False,
                                  use_tc_tiling_on_sc=False)


def _worker_ids():
    c = lax.axis_index("c")
    s = lax.axis_index("s")
    return c, s, c * NS + s


def _writeback(c, s, acc, out0_hbm, out1_hbm):
    @pl.when(jnp.logical_and(s == 0, c == 0))
    def _():
        pltpu.sync_copy(acc, out0_hbm)

    @pl.when(jnp.logical_and(s == 0, c == 1))
    def _():
        pltpu.sync_copy(acc, out1_hbm)


DEG_Q = 8


def _deg_body(ew_hbm, row2d_hbm, zn_hbm, out0_hbm, out1_hbm, wall, rall, acc,
              dsem):
    c, s, w = _worker_ids()

    @pl.when(s == 0)
    def _():
        pltpu.sync_copy(zn_hbm, acc)

    pltpu.sync_copy(ew_hbm.at[pl.ds(w * EPW, EPW)], wall)
    pltpu.sync_copy(row2d_hbm.at[w], rall)
    plsc.subcore_barrier()

    def fire(k):
        pltpu.make_async_copy(
            wall.at[pl.ds(k * CH, CH)], acc.at[rall.at[k]], dsem
        ).start(add=True)

    def drain(k):
        pltpu.make_async_copy(
            wall.at[pl.ds(k * CH, CH)], acc.at[rall.at[k]], dsem
        ).wait()

    for k in range(DEG_Q):
        fire(k)

    def step(k, carry):
        drain(k - DEG_Q)
        fire(k)
        return carry

    lax.fori_loop(DEG_Q, NCHUNK, step, 0)
    for k in range(DEG_Q):
        drain(k)
    plsc.subcore_barrier()
    _writeback(c, s, acc, out0_hbm, out1_hbm)


def _deg_call(ew, row2d, zn):
    return pl.kernel(
        _deg_body,
        out_type=(jax.ShapeDtypeStruct((N,), jnp.float32),
                  jax.ShapeDtypeStruct((N,), jnp.float32)),
        mesh=plsc.VectorSubcoreMesh(**_MESH),
        compiler_params=_SC_PARAMS,
        scratch_types=[
            pltpu.VMEM((EPW,), jnp.float32),
            pltpu.VMEM((NCHUNK, CH), jnp.int32),
            pltpu.MemorySpace.VMEM_SHARED((N,), jnp.float32),
            pltpu.SemaphoreType.DMA,
        ],
    )(ew, row2d, zn)


NCH = 2000
NNCH = EPW // NCH


def _norm_body(row_hbm, col_hbm, ew_hbm, dinv_hbm, out_hbm,
               dbuf, rbuf, cbuf, wbuf, nbuf, sem):
    _, _, w = _worker_ids()
    pltpu.sync_copy(dinv_hbm, dbuf)

    def step(k, carry):
        off = w * EPW + k * NCH
        pltpu.sync_copy(row_hbm.at[pl.ds(off, NCH)], rbuf)
        pltpu.sync_copy(col_hbm.at[pl.ds(off, NCH)], cbuf)
        pltpu.sync_copy(ew_hbm.at[pl.ds(off, NCH)], wbuf)

        def grp(g, carry2):
            sl = pl.ds(g * 16, 16)
            dr = plsc.load_gather(dbuf, [rbuf[sl]])
            dc = plsc.load_gather(dbuf, [cbuf[sl]])
            nbuf[sl] = -(dr * wbuf[sl] * dc)
            return carry2

        lax.fori_loop(0, NCH // 16, grp, 0)
        pltpu.sync_copy(nbuf, out_hbm.at[pl.ds(off, NCH)])
        return carry

    lax.fori_loop(0, NNCH, step, 0)


def _norm_call(row, col, ew, dinv):
    return pl.kernel(
        _norm_body,
        out_type=jax.ShapeDtypeStruct((E,), jnp.float32),
        mesh=plsc.VectorSubcoreMesh(**_MESH),
        compiler_params=_SC_PARAMS,
        scratch_types=[
            pltpu.VMEM((N,), jnp.float32),
            pltpu.VMEM((NCH,), jnp.int32),
            pltpu.VMEM((NCH,), jnp.int32),
            pltpu.VMEM((NCH,), jnp.float32),
            pltpu.VMEM((NCH,), jnp.float32),
            pltpu.SemaphoreType.DMA,
        ],
    )(row, col, ew, dinv)


def _seg_body(F, xin_hbm, row2d_hbm, col2d_hbm, norm_hbm, z_hbm,
              out0_hbm, out1_hbm, rall, call, nall, dbuf0, dbuf1, acc,
              gsem0, gsem1, ssem0, ssem1):
    c, s, w = _worker_ids()

    @pl.when(s == 0)
    def _():
        pltpu.sync_copy(z_hbm, acc)

    pltpu.sync_copy(row2d_hbm.at[w], rall)
    pltpu.sync_copy(col2d_hbm.at[w], call)
    pltpu.sync_copy(norm_hbm.at[pl.ds(w * EPW, EPW)], nall)
    plsc.subcore_barrier()

    def g_start(k, dbuf, gsem):
        pltpu.make_async_copy(xin_hbm.at[rall.at[k]], dbuf, gsem).start()

    def g_wait(k, dbuf, gsem):
        pltpu.make_async_copy(xin_hbm.at[rall.at[k]], dbuf, gsem).wait()

    def s_start(k, dbuf, ssem):
        pltpu.make_async_copy(dbuf, acc.at[call.at[k]], ssem).start(add=True)

    def s_wait(k, dbuf, ssem):
        pltpu.make_async_copy(dbuf, acc.at[call.at[k]], ssem).wait()

    def scale(k, dbuf):
        @plsc.parallel_loop(0, CH // 16, 1, unroll=2)
        def _(g):
            n16 = nall[pl.ds(k * CH + g * 16, 16)]
            for j in range(16):
                spl = jnp.take_along_axis(
                    n16, jnp.full((16,), j, jnp.int32), axis=0)
                e = g * 16 + j
                for v in range(F // 16):
                    fs = pl.ds(v * 16, 16)
                    dbuf[e, fs] = dbuf[e, fs] * spl

    g_start(0, dbuf0, gsem0)

    def pair(p, carry):
        k0 = 2 * p
        @pl.when(p > 0)
        def _():
            s_wait(k0 - 1, dbuf1, ssem1)
        g_start(k0 + 1, dbuf1, gsem1)
        g_wait(k0, dbuf0, gsem0)
        scale(k0, dbuf0)
        s_start(k0, dbuf0, ssem0)
        s_wait(k0, dbuf0, ssem0)
        g_start(k0 + 2, dbuf0, gsem0)
        g_wait(k0 + 1, dbuf1, gsem1)
        scale(k0 + 1, dbuf1)
        s_start(k0 + 1, dbuf1, ssem1)
        return carry

    lax.fori_loop(0, (NCHUNK - 1) // 2, pair, 0)
    kl = NCHUNK - 1
    s_wait(kl - 1, dbuf1, ssem1)
    g_wait(kl, dbuf0, gsem0)
    scale(kl, dbuf0)
    s_start(kl, dbuf0, ssem0)
    s_wait(kl, dbuf0, ssem0)
    plsc.subcore_barrier()
    _writeback(c, s, acc, out0_hbm, out1_hbm)


def _seg_call(F, xin, row2d, col2d, norm, zeros2d):
    return pl.kernel(
        functools.partial(_seg_body, F),
        out_type=(jax.ShapeDtypeStruct((N, F), jnp.float32),
                  jax.ShapeDtypeStruct((N, F), jnp.float32)),
        mesh=plsc.VectorSubcoreMesh(**_MESH),
        compiler_params=_SC_PARAMS,
        scratch_types=[
            pltpu.VMEM((NCHUNK, CH), jnp.int32),
            pltpu.VMEM((NCHUNK, CH), jnp.int32),
            pltpu.VMEM((EPW,), jnp.float32),
            pltpu.VMEM((CH, F), jnp.float32),
            pltpu.VMEM((CH, F), jnp.float32),
            pltpu.MemorySpace.VMEM_SHARED((N, F), jnp.float32),
            pltpu.SemaphoreType.DMA,
            pltpu.SemaphoreType.DMA,
            pltpu.SemaphoreType.DMA,
            pltpu.SemaphoreType.DMA,
        ],
    )(xin, row2d, col2d, norm, zeros2d)


def _dinv_body(p0_ref, p1_ref, out_ref):
    deg = p0_ref[...] + p1_ref[...]
    out_ref[...] = jnp.where(
        deg > 0.0, lax.rsqrt(jnp.maximum(deg, 1e-12)), 0.0)


def _dinv_call(p0, p1):
    return pl.pallas_call(
        _dinv_body,
        out_shape=jax.ShapeDtypeStruct((1, N), jnp.float32),
    )(p0.reshape(1, N), p1.reshape(1, N))


MBLK = 2000


def _dense_body(head, x_ref, p0_ref, p1_ref, wz0_ref, wz1_ref, wh0_ref,
                wh1_ref, bz_ref, bh_ref, lw_ref, lb_ref, out_ref):
    t = p0_ref[...] + p1_ref[...]
    x = x_ref[...]
    a = (jnp.dot(x, wz0_ref[...], preferred_element_type=jnp.float32)
         + jnp.dot(t, wz1_ref[...], preferred_element_type=jnp.float32)
         + bz_ref[...])
    b = (jnp.dot(x, wh0_ref[...], preferred_element_type=jnp.float32)
         + jnp.dot(t, wh1_ref[...], preferred_element_type=jnp.float32)
         + bh_ref[...])
    h = (1.0 - jax.nn.sigmoid(a)) * jnp.tanh(b)
    if head:
        out_ref[...] = (jnp.dot(jax.nn.relu(h), lw_ref[...],
                                preferred_element_type=jnp.float32)
                        + lb_ref[...])
    else:
        out_ref[...] = h


def _dense_call(F, head, x, p0, p1, wz0, wz1, wh0, wh1, bz, bh, lw, lb):
    fout = 1 if head else HID
    return pl.pallas_call(
        functools.partial(_dense_body, head),
        grid=(N // MBLK,),
        in_specs=[
            pl.BlockSpec((MBLK, F), lambda i: (i, 0)),
            pl.BlockSpec((MBLK, F), lambda i: (i, 0)),
            pl.BlockSpec((MBLK, F), lambda i: (i, 0)),
            pl.BlockSpec((F, HID), lambda i: (0, 0)),
            pl.BlockSpec((F, HID), lambda i: (0, 0)),
            pl.BlockSpec((F, HID), lambda i: (0, 0)),
            pl.BlockSpec((F, HID), lambda i: (0, 0)),
            pl.BlockSpec((1, HID), lambda i: (0, 0)),
            pl.BlockSpec((1, HID), lambda i: (0, 0)),
            pl.BlockSpec((HID, 1), lambda i: (0, 0)),
            pl.BlockSpec((1, 1), lambda i: (0, 0)),
        ],
        out_specs=pl.BlockSpec((MBLK, fout), lambda i: (i, 0)),
        out_shape=jax.ShapeDtypeStruct((N, fout), jnp.float32),
    )(x, p0, p1, wz0, wz1, wh0, wh1, bz, bh, lw, lb)


def kernel(x, edge_index, edge_weight, W0x0, W0x1, b0x, W0h0, W0h1, b0h,
           W1x0, W1x1, b1x, W1h0, W1h1, b1h, lin_w, lin_b):
    row = edge_index[0].astype(jnp.int32)
    col = edge_index[1].astype(jnp.int32)
    ew = edge_weight.astype(jnp.float32)

    zn = jnp.zeros((N,), jnp.float32)
    z128 = jnp.zeros((N, F_IN), jnp.float32)
    z64 = jnp.zeros((N, HID), jnp.float32)

    row2d = row.reshape(NW, NCHUNK, CH)
    col2d = col.reshape(NW, NCHUNK, CH)

    d0, d1 = _deg_call(ew, row2d, zn)
    dinv = _dinv_call(d0, d1).reshape(N)
    norm = _norm_call(row, col, ew, dinv)

    bz0 = (b0x[0] + b0h[0]).reshape(1, HID)
    bh0 = (b0x[2] + b0h[2]).reshape(1, HID)
    bz1 = (b1x[0] + b1h[0]).reshape(1, HID)
    bh1 = (b1x[2] + b1h[2]).reshape(1, HID)
    lb = lin_b.reshape(1, 1)

    t0a, t0b = _seg_call(F_IN, x, row2d, col2d, norm, z128)
    h0 = _dense_call(F_IN, False, x, t0a, t0b, W0x0[0], W0x1[0], W0x0[2],
                     W0x1[2], bz0, bh0, lin_w, lb)
    t1a, t1b = _seg_call(HID, h0, row2d, col2d, norm, z64)
    out = _dense_call(HID, True, h0, t1a, t1b, W1x0[0], W1x1[0], W1x0[2],
                      W1x1[2], bz1, bh1, lin_w, lb)
    return out
